# baseline (speedup 1.0000x reference)
import jax
import jax.numpy as jnp
from jax.experimental import pallas as pl
from jax.experimental.pallas import tpu as pltpu

_B, _H, _W, _C = 8, 224, 224, 384
_NROWS = _B * _H * _W
_ROW = 6 * _W + 6
_COL = 1
_RING = 8
_LAG = 3
_CHUNK = 4096

_SIZES = [512, 1024, 2048] + [_CHUNK] * 96 + [1024, 2048, 1024, 512]
assert sum(_SIZES) == _NROWS
_STARTS = [sum(_SIZES[:i]) for i in range(len(_SIZES))]
_N = len(_SIZES)

_TCHUNK = next(i for i in range(_N)
               if _STARTS[i] <= _ROW < _STARTS[i] + _SIZES[i])
_TOFF = _ROW - _STARTS[_TCHUNK]
_TOFF8 = (_TOFF // 8) * 8


def _body(x_hbm, o_hbm, *rest):
    bufs = rest[:_RING]
    in_sems, out_sems = rest[_RING], rest[_RING + 1]
    in_copies = [None] * _N
    out_copies = [None] * _N
    out_waited = [False] * _N

    def start_in(i):
        b = i % _RING
        c = pltpu.make_async_copy(
            x_hbm.at[pl.ds(_STARTS[i], _SIZES[i]), :],
            bufs[b].at[pl.ds(0, _SIZES[i]), :], in_sems.at[b])
        c.start()
        in_copies[i] = c

    for i in range(min(_RING, _N)):
        start_in(i)
    for i in range(_N):
        b = i % _RING
        in_copies[i].wait()
        if i == _TCHUNK:
            r = jax.lax.broadcasted_iota(jnp.int32, (8, _C), 0)
            c2 = jax.lax.broadcasted_iota(jnp.int32, (8, _C), 1)
            hit = (r == (_TOFF - _TOFF8)) & (c2 == _COL)
            tile = bufs[b][pl.ds(_TOFF8, 8), :]
            bufs[b][pl.ds(_TOFF8, 8), :] = jnp.where(
                hit, jnp.float32(1.0), tile)
        oc = pltpu.make_async_copy(
            bufs[b].at[pl.ds(0, _SIZES[i]), :],
            o_hbm.at[pl.ds(_STARTS[i], _SIZES[i]), :], out_sems.at[b])
        oc.start()
        out_copies[i] = oc
        j = i - _LAG
        if j >= 0 and j + _RING < _N:
            out_copies[j].wait()
            out_waited[j] = True
            start_in(j + _RING)
    for i in range(_N):
        if not out_waited[i]:
            out_copies[i].wait()


def kernel(x):
    xf = x.reshape(_NROWS, _C)
    out = pl.pallas_call(
        _body,
        in_specs=[pl.BlockSpec(memory_space=pl.ANY)],
        out_specs=pl.BlockSpec(memory_space=pl.ANY),
        out_shape=jax.ShapeDtypeStruct((_NROWS, _C), jnp.float32),
        scratch_shapes=(
            [pltpu.VMEM((_CHUNK, _C), jnp.float32) for _ in range(_RING)]
            + [pltpu.SemaphoreType.DMA((_RING,)),
               pltpu.SemaphoreType.DMA((_RING,))]
        ),
    )(xf)
    return out.reshape(_B, _H, _W, _C)
